# baseline (speedup 1.0000x reference)
import functools

import jax
import jax.numpy as jnp
from jax import lax
from jax.experimental import pallas as pl
from jax.experimental.pallas import tpu as pltpu
from jax.experimental.pallas import tpu_sc as plsc

N_CENT = 1024
DIM = 64
N_ROWS = 9216
BLK = 512
NBLK = N_ROWS // BLK
DECAY = 0.99

SC_CORES = 2
SC_SUBCORES = 16
NW = SC_CORES * SC_SUBCORES
RPW = N_ROWS // NW
GCH = 96
NGCH = RPW // GCH
HROWS = N_ROWS // SC_SUBCORES
HCH = HROWS // GCH
EROWS = N_CENT // SC_SUBCORES
IDX2D = N_ROWS // GCH


def _rowsum64(y):
    acc = y[:, 0:8]
    for t in range(1, 8):
        acc = acc + y[:, t * 8:(t + 1) * 8]
    b = acc[:, :4] + acc[:, 4:]
    c = b[:, :2] + b[:, 2:]
    return c[:, 0:1] + c[:, 1:2]


def _colsum64(y):
    acc = y[0:8, :]
    for t in range(1, 8):
        acc = acc + y[t * 8:(t + 1) * 8, :]
    b = acc[:4, :] + acc[4:, :]
    c = b[:2, :] + b[2:, :]
    return c[0:1, :] + c[1:2, :]


def _tc_body(z_ref, cbt_ref, idx_ref):
    zb = z_ref[...]
    cbt = cbt_ref[...]
    m = jnp.dot(zb, cbt, preferred_element_type=jnp.float32)
    sumz = _rowsum64(zb * zb)
    sumc = _colsum64(cbt * cbt)
    d = sumz - 2.0 * m + sumc
    dmin = jnp.min(d, axis=1, keepdims=True)
    ii = lax.broadcasted_iota(jnp.int32, d.shape, 1)
    idx = jnp.min(jnp.where(d == dmin, ii, N_CENT), axis=1)
    idx_ref[...] = idx


def _tc_argmin(flat, cbt):
    return pl.pallas_call(
        _tc_body,
        grid=(NBLK,),
        in_specs=[
            pl.BlockSpec((BLK, DIM), lambda i: (i, 0)),
            pl.BlockSpec((DIM, N_CENT), lambda i: (0, 0)),
        ],
        out_specs=pl.BlockSpec((BLK,), lambda i: (i,)),
        out_shape=jax.ShapeDtypeStruct((N_ROWS,), jnp.int32),
    )(flat, cbt)


@functools.partial(
    pl.kernel,
    out_type=[
        jax.ShapeDtypeStruct((N_ROWS, DIM), jnp.float32),
        jax.ShapeDtypeStruct((N_ROWS, DIM), jnp.float32),
        jax.ShapeDtypeStruct((N_CENT,), jnp.float32),
    ],
    mesh=plsc.VectorSubcoreMesh(core_axis_name="c", subcore_axis_name="s"),
    compiler_params=pltpu.CompilerParams(use_tc_tiling_on_sc=False),
    scratch_types=[
        pltpu.VMEM((RPW,), jnp.int32),
        pltpu.VMEM((RPW, DIM), jnp.float32),
        pltpu.VMEM((GCH, 16), jnp.float32),
        pltpu.VMEM((HROWS,), jnp.int32),
        pltpu.VMEM((EROWS, 16), jnp.float32),
        pltpu.VMEM((EROWS, 16), jnp.float32),
        pltpu.VMEM((EROWS,), jnp.float32),
        pltpu.VMEM((EROWS,), jnp.float32),
        pltpu.VMEM_SHARED((N_CENT, 16), jnp.float32),
        pltpu.SemaphoreType.DMA,
    ],
)
def _sc_quantize(idx2d_hbm, cb_hbm, cc_hbm, ones_hbm, zeros_hbm,
                 q_hbm, q2_hbm, counts_hbm,
                 idx3_v, rows_v, ones_v, idx2_v, z_v, hv, cc_v, outc_v,
                 hist_sh, sem):
    cid = lax.axis_index("c")
    sid = lax.axis_index("s")
    wid = sid * SC_CORES + cid
    base = wid * RPW

    pltpu.sync_copy(idx2d_hbm.at[pl.ds(base, RPW)], idx3_v)
    cps = []
    for j in range(NGCH):
        cps.append(pltpu.async_copy(
            cb_hbm.at[idx3_v.at[pl.ds(j * GCH, GCH)]],
            rows_v.at[pl.ds(j * GCH, GCH)], sem))

    @pl.when(cid == 0)
    def _hist():
        pltpu.sync_copy(zeros_hbm, z_v)
        pltpu.sync_copy(z_v, hist_sh.at[pl.ds(sid * EROWS, EROWS)])
        pltpu.sync_copy(ones_hbm, ones_v)
        pltpu.sync_copy(idx2d_hbm.at[pl.ds(sid * HROWS, HROWS)], idx2_v)
        plsc.subcore_barrier()
        for j in range(HCH):
            pltpu.sync_copy(
                ones_v, hist_sh.at[idx2_v.at[pl.ds(j * GCH, GCH)]], add=True)
        plsc.subcore_barrier()
        pltpu.sync_copy(hist_sh.at[pl.ds(sid * EROWS, EROWS)], hv)
        pltpu.sync_copy(cc_hbm.at[pl.ds(sid * EROWS, EROWS)], cc_v)
        lane = lax.iota(jnp.int32, 16)
        for k in range(EROWS // 16):
            acc = jnp.zeros((16,), jnp.float32)
            for r in range(16):
                row = hv[16 * k + r, :]
                acc = jnp.where(lane == r, row, acc)
            cc16 = cc_v[pl.ds(16 * k, 16)]
            outc_v[pl.ds(16 * k, 16)] = DECAY * cc16 + (1.0 - DECAY) * acc
        pltpu.sync_copy(outc_v, counts_hbm.at[pl.ds(sid * EROWS, EROWS)])

    for cp in cps:
        cp.wait()
    pltpu.sync_copy(rows_v, q_hbm.at[pl.ds(wid * RPW, RPW)])
    pltpu.sync_copy(rows_v, q2_hbm.at[pl.ds(wid * RPW, RPW)])


def kernel(z, codebook, cluster_counts):
    flat = z.reshape(N_ROWS, DIM)
    cbt = codebook.T
    idx1d = _tc_argmin(flat, cbt)
    ones = jnp.ones((GCH, 16), jnp.float32)
    zeros = jnp.zeros((EROWS, 16), jnp.float32)
    q, q2, new_counts = _sc_quantize(
        idx1d, codebook, cluster_counts, ones, zeros)
    return q.reshape(z.shape), new_counts, q2.reshape(z.shape)


# speedup vs baseline: 1.1083x; 1.1083x over previous
import functools

import jax
import jax.numpy as jnp
from jax import lax
from jax.experimental import pallas as pl
from jax.experimental.pallas import tpu as pltpu
from jax.experimental.pallas import tpu_sc as plsc

N_CENT = 1024
DIM = 64
N_ROWS = 9216
BLK = 512
NBLK = N_ROWS // BLK
DECAY = 0.99

SC_CORES = 2
SC_SUBCORES = 16
NW = SC_CORES * SC_SUBCORES
RPW = N_ROWS // NW
GCH = 96
NGCH = RPW // GCH
HROWS = N_ROWS // SC_SUBCORES
HCH = HROWS // GCH
EROWS = N_CENT // SC_SUBCORES


def _rowsum64(y):
    acc = y[:, 0:8]
    for t in range(1, 8):
        acc = acc + y[:, t * 8:(t + 1) * 8]
    b = acc[:, :4] + acc[:, 4:]
    c = b[:, :2] + b[:, 2:]
    return c[:, 0:1] + c[:, 1:2]


def _colsum64(y):
    acc = y[0:8, :]
    for t in range(1, 8):
        acc = acc + y[t * 8:(t + 1) * 8, :]
    b = acc[:4, :] + acc[4:, :]
    c = b[:2, :] + b[2:, :]
    return c[0:1, :] + c[1:2, :]


def _tc_body(z_ref, cbt_ref, idx_ref):
    zb = z_ref[...]
    cbt = cbt_ref[...]
    sumz = _rowsum64(zb * zb)
    sumc = _colsum64(cbt * cbt)
    m = jnp.dot(zb, cbt, preferred_element_type=jnp.float32)
    d = sumz - 2.0 * m + sumc
    dmin = jnp.min(d, axis=1, keepdims=True)
    ii = lax.broadcasted_iota(jnp.int32, d.shape, 1).astype(jnp.float32)
    idxf = jnp.min(jnp.where(d == dmin, ii, float(N_CENT)), axis=1)
    idx_ref[...] = idxf.astype(jnp.int32)


def _tc_argmin(flat, cbt):
    return pl.pallas_call(
        _tc_body,
        grid=(NBLK,),
        in_specs=[
            pl.BlockSpec((BLK, DIM), lambda i: (i, 0)),
            pl.BlockSpec((DIM, N_CENT), lambda i: (0, 0)),
        ],
        out_specs=pl.BlockSpec((BLK,), lambda i: (i,)),
        out_shape=jax.ShapeDtypeStruct((N_ROWS,), jnp.int32),
    )(flat, cbt)


@functools.partial(
    pl.kernel,
    out_type=[
        jax.ShapeDtypeStruct((16, 576, 128), jnp.float32),
        jax.ShapeDtypeStruct((N_CENT,), jnp.float32),
    ],
    mesh=plsc.VectorSubcoreMesh(core_axis_name="c", subcore_axis_name="s"),
    compiler_params=pltpu.CompilerParams(use_tc_tiling_on_sc=False),
    scratch_types=[
        pltpu.VMEM((RPW,), jnp.int32),
        pltpu.VMEM((RPW, 128), jnp.float32),
        pltpu.VMEM((GCH, 16), jnp.float32),
        pltpu.VMEM((HROWS,), jnp.int32),
        pltpu.VMEM((EROWS, 16), jnp.float32),
        pltpu.VMEM((EROWS, 16), jnp.float32),
        pltpu.VMEM((EROWS,), jnp.float32),
        pltpu.VMEM((EROWS,), jnp.float32),
        pltpu.VMEM_SHARED((N_CENT, 16), jnp.float32),
        pltpu.SemaphoreType.DMA,
    ],
)
def _sc_quantize(idx_hbm, cb_hbm, cc_hbm, ones_hbm, zeros_hbm,
                 q_hbm, counts_hbm,
                 idx3_v, rows_v, ones_v, idx2_v, z_v, hv, cc_v, outc_v,
                 hist_sh, sem):
    cid = lax.axis_index("c")
    sid = lax.axis_index("s")
    wid = sid * SC_CORES + cid
    base = wid * RPW

    pltpu.sync_copy(idx_hbm.at[pl.ds(base, RPW)], idx3_v)
    cps = []
    for j in range(NGCH):
        cps.append(pltpu.async_copy(
            cb_hbm.at[idx3_v.at[pl.ds(j * GCH, GCH)]],
            rows_v.at[pl.ds(j * GCH, GCH)], sem))

    @pl.when(cid == 0)
    def _hist():
        pltpu.sync_copy(zeros_hbm, z_v)
        pltpu.sync_copy(z_v, hist_sh.at[pl.ds(sid * EROWS, EROWS)])
        pltpu.sync_copy(ones_hbm, ones_v)
        pltpu.sync_copy(idx_hbm.at[pl.ds(sid * HROWS, HROWS)], idx2_v)
        plsc.subcore_barrier()
        for j in range(HCH):
            pltpu.sync_copy(
                ones_v, hist_sh.at[idx2_v.at[pl.ds(j * GCH, GCH)]], add=True)
        plsc.subcore_barrier()
        pltpu.sync_copy(hist_sh.at[pl.ds(sid * EROWS, EROWS)], hv)
        pltpu.sync_copy(cc_hbm.at[pl.ds(sid * EROWS, EROWS)], cc_v)
        lane = lax.iota(jnp.int32, 16)
        for k in range(EROWS // 16):
            acc = jnp.zeros((16,), jnp.float32)
            for r in range(16):
                row = hv[16 * k + r, :]
                acc = jnp.where(lane == r, row, acc)
            cc16 = cc_v[pl.ds(16 * k, 16)]
            outc_v[pl.ds(16 * k, 16)] = DECAY * cc16 + (1.0 - DECAY) * acc
        pltpu.sync_copy(outc_v, counts_hbm.at[pl.ds(sid * EROWS, EROWS)])

    for cp in cps:
        cp.wait()
    b = wid // 2
    off = (wid % 2) * RPW
    pltpu.sync_copy(rows_v, q_hbm.at[b, pl.ds(off, RPW)])


def kernel(z, codebook, cluster_counts):
    flat = z.reshape(N_ROWS, DIM)
    cbt = codebook.T
    idx1d = _tc_argmin(flat, cbt)
    cb_pad = jnp.pad(codebook, ((0, 0), (0, 128 - DIM)))
    ones = jnp.ones((GCH, 16), jnp.float32)
    zeros = jnp.zeros((EROWS, 16), jnp.float32)
    q, new_counts = _sc_quantize(
        idx1d, cb_pad, cluster_counts, ones, zeros)
    qq = q[:, :, :DIM]
    return qq, new_counts, qq
